# Initial kernel scaffold; baseline (speedup 1.0000x reference)
import jax
import jax.numpy as jnp
from jax.experimental import pallas as pl


def kernel(users, pos_items, neg_items, adj_indices, adj_values, user_emb, item_emb, W_gc_0, b_gc_0, W_bi_0, b_bi_0, W_gc_1, b_gc_1, W_bi_1, b_bi_1):
    raise NotImplementedError("write your pallas kernel here")


# speedup vs baseline: 6.5492x; 6.5492x over previous
import functools

import jax
import jax.numpy as jnp
from jax import lax
from jax.experimental import pallas as pl
from jax.experimental.pallas import tpu as pltpu
from jax.experimental.pallas import tpu_sc as plsc

N_USER = 50000
N_ITEM = 50000
N = N_USER + N_ITEM
D = 32
B = 4096
NB3 = 3 * B
NC, NS = 2, 16
NW = NC * NS
C = 2048
NCHUNK = 50
EW = C * NCHUNK
NNZ_PAD = EW * NS
HALF = N // 2
TBL = HALF
SCAP = C + 128
G = 128
EGO_W = NB3 // NW
OUT_T = NB3 // NS
ACC_R = NB3 + 8


def _sc_body(needed, rowp, colp, valp, emb, neg1, zacc,
             ego_out, sidep_out,
             table, rowb, colb, valb, sslotf, scolf, svalf, sslot2d,
             rowsb, nbuf, slotsb, acc):
    cid = lax.axis_index("c")
    sid = lax.axis_index("s")
    wid = sid * NC + cid
    lo = cid * HALF

    pltpu.sync_copy(neg1, table)

    @pl.when(sid == 0)
    def _():
        pltpu.sync_copy(zacc, acc)

    def build_blk(c, _):
        pltpu.sync_copy(needed.at[pl.ds(c * OUT_T, OUT_T)], nbuf)

        def build_vreg(k, _):
            idx16 = nbuf[pl.ds(k * 16, 16)] - lo
            m = (idx16 >= 0) & (idx16 < HALF)
            idxc = jnp.clip(idx16, 0, HALF - 1)
            pos16 = jnp.full((16,), c * OUT_T + k * 16, jnp.int32) + lax.iota(
                jnp.int32, 16)
            plsc.store_scatter(table, [idxc], pos16, mask=m)
            return 0

        lax.fori_loop(0, OUT_T // 16, build_vreg, 0)
        return 0

    lax.fori_loop(0, NS, build_blk, 0)

    gbase = wid * EGO_W
    pltpu.sync_copy(needed.at[pl.ds(gbase, EGO_W)], nbuf.at[pl.ds(0, EGO_W)])
    for r in range(EGO_W // G):
        pltpu.sync_copy(emb.at[nbuf.at[pl.ds(r * G, G)]], rowsb)
        pltpu.sync_copy(rowsb, ego_out.at[pl.ds(gbase + r * G, G)])

    plsc.subcore_barrier()

    ebase = sid * EW

    def chunk_body(ci, _):
        base = ebase + ci * C
        pltpu.sync_copy(rowp.at[pl.ds(base, C)], rowb)
        pltpu.sync_copy(colp.at[pl.ds(base, C)], colb)
        pltpu.sync_copy(valp.at[pl.ds(base, C)], valb)

        def fbody(i, cnt):
            o = i * 16
            r16 = rowb[pl.ds(o, 16)] - lo
            inr = (r16 >= 0) & (r16 < HALF)
            s16 = plsc.load_gather(table, [jnp.clip(r16, 0, HALF - 1)])
            m = inr & (s16 >= 0)
            plsc.store_compressed(sslotf.at[pl.ds(cnt, 16)], s16, mask=m)
            plsc.store_compressed(scolf.at[pl.ds(cnt, 16)],
                                  colb[pl.ds(o, 16)], mask=m)
            plsc.store_compressed(svalf.at[pl.ds(cnt, 16)],
                                  valb[pl.ds(o, 16)], mask=m)
            return cnt + jnp.max(plsc.all_reduce_population_count(m))

        cnt = lax.fori_loop(0, C // 16, fbody, jnp.int32(0))

        zi = jnp.zeros((16,), jnp.int32)
        zf = jnp.zeros((16,), jnp.float32)
        for k in range(G // 16):
            sslotf[pl.ds(cnt + k * 16, 16)] = zi
            scolf[pl.ds(cnt + k * 16, 16)] = zi
            svalf[pl.ds(cnt + k * 16, 16)] = zf
        mp = (cnt + (G - 1)) // G

        def gbody(g, _):
            for k in range(G // 16):
                sslot2d[g, pl.ds(k * 16, 16)] = sslotf[pl.ds(g * G + k * 16,
                                                             16)]
            pltpu.sync_copy(emb.at[scolf.at[pl.ds(g * G, G)]], rowsb)

            def scale16(b, _):
                v16 = svalf[pl.ds(g * G + b * 16, 16)]
                for j in range(16):
                    i = b * 16 + j
                    v = v16[j]
                    rowsb[i, pl.ds(0, 16)] = rowsb[i, pl.ds(0, 16)] * v
                    rowsb[i, pl.ds(16, 16)] = rowsb[i, pl.ds(16, 16)] * v
                return 0

            lax.fori_loop(0, G // 16, scale16, 0)
            pltpu.sync_copy(rowsb, acc.at[sslot2d.at[g]], add=True)
            return 0

        lax.fori_loop(0, mp, gbody, 0)
        return 0

    lax.fori_loop(0, NCHUNK, chunk_body, 0)

    plsc.subcore_barrier()

    pbase = sid * OUT_T
    pltpu.sync_copy(needed.at[pl.ds(pbase, OUT_T)], nbuf)

    def slot_vreg(k, _):
        i16 = nbuf[pl.ds(k * 16, 16)] - lo
        inr = (i16 >= 0) & (i16 < HALF)
        s16 = plsc.load_gather(table, [jnp.clip(i16, 0, HALF - 1)])
        slotsb[pl.ds(k * 16, 16)] = jnp.where(inr & (s16 >= 0), s16,
                                              jnp.full((16,), NB3, jnp.int32))
        return 0

    lax.fori_loop(0, OUT_T // 16, slot_vreg, 0)
    obase = cid * NB3 + pbase
    for r in range(OUT_T // G):
        pltpu.sync_copy(acc.at[slotsb.at[pl.ds(r * G, G)]], rowsb)
        pltpu.sync_copy(rowsb, sidep_out.at[pl.ds(obase + r * G, G)])


_sc_call = functools.partial(
    pl.kernel,
    out_type=(
        jax.ShapeDtypeStruct((NB3, D), jnp.float32),
        jax.ShapeDtypeStruct((2 * NB3, D), jnp.float32),
    ),
    mesh=plsc.VectorSubcoreMesh(core_axis_name="c", subcore_axis_name="s"),
    compiler_params=pltpu.CompilerParams(needs_layout_passes=False,
                                         use_tc_tiling_on_sc=False),
    scratch_types=[
        pltpu.VMEM((TBL,), jnp.int32),
        pltpu.VMEM((C,), jnp.int32),
        pltpu.VMEM((C,), jnp.int32),
        pltpu.VMEM((C,), jnp.float32),
        pltpu.VMEM((SCAP,), jnp.int32),
        pltpu.VMEM((SCAP,), jnp.int32),
        pltpu.VMEM((SCAP,), jnp.float32),
        pltpu.VMEM((SCAP // G, G), jnp.int32),
        pltpu.VMEM((G, D), jnp.float32),
        pltpu.VMEM((OUT_T,), jnp.int32),
        pltpu.VMEM((OUT_T,), jnp.int32),
        pltpu.VMEM_SHARED((ACC_R, D), jnp.float32),
    ],
)(_sc_body)


def _tc_body(ego_ref, sidep_ref, wg0, bg0, wb0, bb0, wg1, bg1, wb1, bb1,
             u_ref, p_ref, n_ref):
    side = sidep_ref[0:NB3, :] + sidep_ref[NB3:2 * NB3, :]
    e0 = ego_ref[...]

    def layer(ego, wg, bg, wb, bb):
        sum_emb = jnp.dot(side, wg[...],
                          preferred_element_type=jnp.float32) + bg[...]
        bi_emb = jnp.dot(ego * side, wb[...],
                         preferred_element_type=jnp.float32) + bb[...]
        x = sum_emb + bi_emb
        x = jnp.where(x >= 0, x, 0.2 * x)
        nrm = jnp.sqrt(jnp.sum(x * x, axis=1, keepdims=True))
        return x, x / jnp.maximum(nrm, 1e-12)

    e1, n1 = layer(e0, wg0, bg0, wb0, bb0)
    e2, n2 = layer(e1, wg1, bg1, wb1, bb1)
    allemb = jnp.concatenate([e0, n1, n2], axis=1)
    u_ref[...] = allemb[0:B]
    p_ref[...] = allemb[B:2 * B]
    n_ref[...] = allemb[2 * B:3 * B]


def _tc_epilogue(ego, sidep, wg0, bg0, wb0, bb0, wg1, bg1, wb1, bb1):
    return pl.pallas_call(
        _tc_body,
        out_shape=[jax.ShapeDtypeStruct((B, 3 * D), jnp.float32)] * 3,
    )(ego, sidep, wg0, bg0, wb0, bb0, wg1, bg1, wb1, bb1)


def kernel(users, pos_items, neg_items, adj_indices, adj_values, user_emb,
           item_emb, W_gc_0, b_gc_0, W_bi_0, b_bi_0, W_gc_1, b_gc_1, W_bi_1,
           b_bi_1):
    needed = jnp.concatenate(
        [users, pos_items + N_USER, neg_items + N_USER]).astype(jnp.int32)
    emb = jnp.concatenate([user_emb, item_emb], axis=0)
    nnz = adj_values.shape[0]
    pad = NNZ_PAD - nnz
    rowp = jnp.pad(adj_indices[0], (0, pad))
    colp = jnp.pad(adj_indices[1], (0, pad))
    valp = jnp.pad(adj_values, (0, pad))
    neg1 = jnp.full((TBL,), -1, jnp.int32)
    zacc = jnp.zeros((ACC_R, D), jnp.float32)
    ego, sidep = _sc_call(needed, rowp, colp, valp, emb, neg1, zacc)
    return _tc_epilogue(ego, sidep, W_gc_0, b_gc_0, W_bi_0, b_bi_0,
                        W_gc_1, b_gc_1, W_bi_1, b_bi_1)


# speedup vs baseline: 21.4180x; 3.2703x over previous
import functools

import jax
import jax.numpy as jnp
from jax import lax
from jax.experimental import pallas as pl
from jax.experimental.pallas import tpu as pltpu
from jax.experimental.pallas import tpu_sc as plsc

N_USER = 50000
N_ITEM = 50000
N = N_USER + N_ITEM
D = 32
B = 4096
NB3 = 3 * B
NC, NS = 2, 16
NW = NC * NS
C = 2048
NCHUNK = 50
EW = C * NCHUNK
NNZ_PAD = EW * NS
HALF = N // 2
TBL = HALF
SCAP0 = 16384
SCAPB = SCAP0 + 128
G = 128
SSH = 14
EGO_W = NB3 // NW
OUT_T = NB3 // NS
ACC_R = NB3 + 8


def _sc_body(needed, rowp, colp, valp, emb, neg1, zacc,
             ego_out, sidep_out,
             table, ebr, ebc, ebv, spack, sval, scolg, sslot2d,
             rowsb, nbuf, slotsb, acc, esem, gsem, ssem):
    cid = lax.axis_index("c")
    sid = lax.axis_index("s")
    wid = sid * NC + cid
    lo = cid * HALF

    pltpu.sync_copy(neg1, table)

    @pl.when(sid == 0)
    def _():
        pltpu.sync_copy(zacc, acc)

    def build_blk(c, _):
        pltpu.sync_copy(needed.at[pl.ds(c * OUT_T, OUT_T)], nbuf)

        def build_vreg(k, _):
            idx16 = nbuf[pl.ds(k * 16, 16)] - lo
            m = (idx16 >= 0) & (idx16 < HALF)
            idxc = jnp.clip(idx16, 0, HALF - 1)
            pos16 = jnp.full((16,), c * OUT_T + k * 16, jnp.int32) + lax.iota(
                jnp.int32, 16)
            plsc.store_scatter(table, [idxc], pos16, mask=m)
            return 0

        lax.fori_loop(0, OUT_T // 16, build_vreg, 0)
        return 0

    lax.fori_loop(0, NS, build_blk, 0)

    gbase = wid * EGO_W
    pltpu.sync_copy(needed.at[pl.ds(gbase, EGO_W)], nbuf.at[pl.ds(0, EGO_W)])
    for r in range(EGO_W // G):
        pltpu.sync_copy(emb.at[nbuf.at[pl.ds(r * G, G)]],
                        rowsb.at[pl.ds(0, G)])
        pltpu.sync_copy(rowsb.at[pl.ds(0, G)],
                        ego_out.at[pl.ds(gbase + r * G, G)])

    plsc.subcore_barrier()

    ebase = sid * EW

    def fire_edges(ci):
        b = (ci & 1) * C
        base = ebase + ci * C
        pltpu.async_copy(rowp.at[pl.ds(base, C)], ebr.at[pl.ds(b, C)], esem)
        pltpu.async_copy(colp.at[pl.ds(base, C)], ebc.at[pl.ds(b, C)], esem)
        pltpu.async_copy(valp.at[pl.ds(base, C)], ebv.at[pl.ds(b, C)], esem)

    def wait_edges(ci):
        b = (ci & 1) * C
        base = ebase + ci * C
        pltpu.make_async_copy(rowp.at[pl.ds(base, C)],
                              ebr.at[pl.ds(b, C)], esem).wait()
        pltpu.make_async_copy(colp.at[pl.ds(base, C)],
                              ebc.at[pl.ds(b, C)], esem).wait()
        pltpu.make_async_copy(valp.at[pl.ds(base, C)],
                              ebv.at[pl.ds(b, C)], esem).wait()

    fire_edges(jnp.int32(0))

    def chunk_body(ci, cnt):
        wait_edges(ci)

        @pl.when(ci + 1 < NCHUNK)
        def _():
            fire_edges(ci + 1)

        b = (ci & 1) * C

        def fbody(i, cnt):
            o = b + i * 16
            r16 = ebr[pl.ds(o, 16)] - lo
            inr = (r16 >= 0) & (r16 < HALF)
            s16 = plsc.load_gather(table, [jnp.clip(r16, 0, HALF - 1)])
            m = inr & (s16 >= 0)
            packed = (ebc[pl.ds(o, 16)] << SSH) | (s16 & (2**SSH - 1))
            plsc.store_compressed(spack.at[pl.ds(cnt, 16)], packed, mask=m)
            plsc.store_compressed(sval.at[pl.ds(cnt, 16)],
                                  ebv[pl.ds(o, 16)], mask=m)
            pc = plsc.all_reduce_population_count(m)
            return jnp.minimum(cnt + pc[0], SCAP0)

        return lax.fori_loop(0, C // 16, fbody, cnt)

    cnt = lax.fori_loop(0, NCHUNK, chunk_body, jnp.int32(0))

    zi = jnp.zeros((16,), jnp.int32)
    zf = jnp.zeros((16,), jnp.float32)
    for k in range(G // 16):
        spack[pl.ds(cnt + k * 16, 16)] = zi
        sval[pl.ds(cnt + k * 16, 16)] = zf
    nb = (cnt + (G - 1)) // G

    def stage(g):
        gb = (g & 1) * G
        for k in range(8):
            p16 = spack[pl.ds(g * G + k * 16, 16)]
            scolg[pl.ds(gb + k * 16, 16)] = p16 >> SSH
            sslot2d[g & 1, pl.ds(k * 16, 16)] = p16 & (2**SSH - 1)

    def fire_gather(g):
        gb = (g & 1) * G
        pltpu.async_copy(emb.at[scolg.at[pl.ds(gb, G)]],
                         rowsb.at[pl.ds(gb, G)], gsem)

    def wait_gather(g):
        gb = (g & 1) * G
        pltpu.make_async_copy(emb.at[scolg.at[pl.ds(gb, G)]],
                              rowsb.at[pl.ds(gb, G)], gsem).wait()

    def fire_scatter(g):
        gb = (g & 1) * G
        pltpu.async_copy(rowsb.at[pl.ds(gb, G)],
                         acc.at[sslot2d.at[g & 1]], ssem, add=True)

    def wait_scatter(g):
        gb = (g & 1) * G
        pltpu.make_async_copy(rowsb.at[pl.ds(gb, G)],
                              acc.at[sslot2d.at[g & 1]], ssem).wait()

    @pl.when(nb > 0)
    def _():
        stage(jnp.int32(0))
        fire_gather(jnp.int32(0))

    def gloop(g, _):
        gb = (g & 1) * G
        wait_gather(g)

        @pl.when(g + 1 < nb)
        def _():
            @pl.when(g >= 1)
            def _():
                wait_scatter(g - 1)

            stage(g + 1)
            fire_gather(g + 1)

        def scale16(bi, _):
            v16 = sval[pl.ds(g * G + bi * 16, 16)]
            for j in range(16):
                i = gb + bi * 16 + j
                v = v16[j]
                rowsb[i, pl.ds(0, 16)] = rowsb[i, pl.ds(0, 16)] * v
                rowsb[i, pl.ds(16, 16)] = rowsb[i, pl.ds(16, 16)] * v
            return 0

        lax.fori_loop(0, G // 16, scale16, 0)
        fire_scatter(g)
        return 0

    lax.fori_loop(0, nb, gloop, 0)

    @pl.when(nb >= 2)
    def _():
        wait_scatter(nb - 2)

    @pl.when(nb >= 1)
    def _():
        wait_scatter(nb - 1)

    plsc.subcore_barrier()

    pbase = sid * OUT_T
    pltpu.sync_copy(needed.at[pl.ds(pbase, OUT_T)], nbuf)

    def slot_vreg(k, _):
        i16 = nbuf[pl.ds(k * 16, 16)] - lo
        inr = (i16 >= 0) & (i16 < HALF)
        s16 = plsc.load_gather(table, [jnp.clip(i16, 0, HALF - 1)])
        slotsb[pl.ds(k * 16, 16)] = jnp.where(inr & (s16 >= 0), s16,
                                              jnp.full((16,), NB3, jnp.int32))
        return 0

    lax.fori_loop(0, OUT_T // 16, slot_vreg, 0)
    obase = cid * NB3 + pbase
    for r in range(OUT_T // G):
        pltpu.sync_copy(acc.at[slotsb.at[pl.ds(r * G, G)]],
                        rowsb.at[pl.ds(0, G)])
        pltpu.sync_copy(rowsb.at[pl.ds(0, G)],
                        sidep_out.at[pl.ds(obase + r * G, G)])


_sc_call = functools.partial(
    pl.kernel,
    out_type=(
        jax.ShapeDtypeStruct((NB3, D), jnp.float32),
        jax.ShapeDtypeStruct((2 * NB3, D), jnp.float32),
    ),
    mesh=plsc.VectorSubcoreMesh(core_axis_name="c", subcore_axis_name="s"),
    compiler_params=pltpu.CompilerParams(needs_layout_passes=False,
                                         use_tc_tiling_on_sc=False),
    scratch_types=[
        pltpu.VMEM((TBL,), jnp.int32),
        pltpu.VMEM((2 * C,), jnp.int32),
        pltpu.VMEM((2 * C,), jnp.int32),
        pltpu.VMEM((2 * C,), jnp.float32),
        pltpu.VMEM((SCAPB,), jnp.int32),
        pltpu.VMEM((SCAPB,), jnp.float32),
        pltpu.VMEM((2 * G,), jnp.int32),
        pltpu.VMEM((2, G), jnp.int32),
        pltpu.VMEM((2 * G, D), jnp.float32),
        pltpu.VMEM((OUT_T,), jnp.int32),
        pltpu.VMEM((OUT_T,), jnp.int32),
        pltpu.VMEM_SHARED((ACC_R, D), jnp.float32),
        pltpu.SemaphoreType.DMA,
        pltpu.SemaphoreType.DMA,
        pltpu.SemaphoreType.DMA,
    ],
)(_sc_body)


def _tc_body(ego_ref, sidep_ref, wg0, bg0, wb0, bb0, wg1, bg1, wb1, bb1,
             u_ref, p_ref, n_ref):
    side = sidep_ref[0:NB3, :] + sidep_ref[NB3:2 * NB3, :]
    e0 = ego_ref[...]

    def layer(ego, wg, bg, wb, bb):
        sum_emb = jnp.dot(side, wg[...],
                          preferred_element_type=jnp.float32) + bg[...]
        bi_emb = jnp.dot(ego * side, wb[...],
                         preferred_element_type=jnp.float32) + bb[...]
        x = sum_emb + bi_emb
        x = jnp.where(x >= 0, x, 0.2 * x)
        nrm = jnp.sqrt(jnp.sum(x * x, axis=1, keepdims=True))
        return x, x / jnp.maximum(nrm, 1e-12)

    e1, n1 = layer(e0, wg0, bg0, wb0, bb0)
    e2, n2 = layer(e1, wg1, bg1, wb1, bb1)
    allemb = jnp.concatenate([e0, n1, n2], axis=1)
    u_ref[...] = allemb[0:B]
    p_ref[...] = allemb[B:2 * B]
    n_ref[...] = allemb[2 * B:3 * B]


def _tc_epilogue(ego, sidep, wg0, bg0, wb0, bb0, wg1, bg1, wb1, bb1):
    return pl.pallas_call(
        _tc_body,
        out_shape=[jax.ShapeDtypeStruct((B, 3 * D), jnp.float32)] * 3,
    )(ego, sidep, wg0, bg0, wb0, bb0, wg1, bg1, wb1, bb1)


def kernel(users, pos_items, neg_items, adj_indices, adj_values, user_emb,
           item_emb, W_gc_0, b_gc_0, W_bi_0, b_bi_0, W_gc_1, b_gc_1, W_bi_1,
           b_bi_1):
    needed = jnp.concatenate(
        [users, pos_items + N_USER, neg_items + N_USER]).astype(jnp.int32)
    emb = jnp.concatenate([user_emb, item_emb], axis=0)
    nnz = adj_values.shape[0]
    pad = NNZ_PAD - nnz
    rowp = jnp.pad(adj_indices[0], (0, pad))
    colp = jnp.pad(adj_indices[1], (0, pad))
    valp = jnp.pad(adj_values, (0, pad))
    neg1 = jnp.full((TBL,), -1, jnp.int32)
    zacc = jnp.zeros((ACC_R, D), jnp.float32)
    ego, sidep = _sc_call(needed, rowp, colp, valp, emb, neg1, zacc)
    return _tc_epilogue(ego, sidep, W_gc_0, b_gc_0, W_bi_0, b_bi_0,
                        W_gc_1, b_gc_1, W_bi_1, b_bi_1)


# speedup vs baseline: 24.7231x; 1.1543x over previous
import functools

import jax
import jax.numpy as jnp
from jax import lax
from jax.experimental import pallas as pl
from jax.experimental.pallas import tpu as pltpu
from jax.experimental.pallas import tpu_sc as plsc

N_USER = 50000
N_ITEM = 50000
N = N_USER + N_ITEM
D = 32
B = 4096
NB3 = 3 * B
NC, NS = 2, 16
NW = NC * NS
C = 2000
NCHUNK = 50
EW = C * NCHUNK
NNZ = 1600000
HALF = N // 2
TBL = HALF
SCAP0 = 8320
SCAPB = SCAP0 + 128
G = 128
SSH = 14
EGO_W = NB3 // NW
OUT_T = NB3 // NS
ACC_R = NB3 + 8


def _sc_body(needed, adj, vals, uemb, iemb, neg1, zacc,
             ego_out, sidep_out,
             table, ebr, ebc, ebv, spacku, svalu, spacki, svali,
             scolg, sslot2d, rowsb, nbuf, slotsb, acc, esem, gsem, ssem):
    cid = lax.axis_index("c")
    sid = lax.axis_index("s")
    wid = sid * NC + cid
    lo = cid * HALF

    pltpu.sync_copy(neg1, table)

    @pl.when(sid == 0)
    def _():
        pltpu.sync_copy(zacc, acc)

    def build_blk(c, _):
        pltpu.sync_copy(needed.at[pl.ds(c * OUT_T, OUT_T)], nbuf)

        def build_vreg(k, _):
            idx16 = nbuf[pl.ds(k * 16, 16)] - lo
            m = (idx16 >= 0) & (idx16 < HALF)
            idxc = jnp.clip(idx16, 0, HALF - 1)
            pos16 = jnp.full((16,), c * OUT_T + k * 16, jnp.int32) + lax.iota(
                jnp.int32, 16)
            plsc.store_scatter(table, [idxc], pos16, mask=m)
            return 0

        lax.fori_loop(0, OUT_T // 16, build_vreg, 0)
        return 0

    lax.fori_loop(0, NS, build_blk, 0)

    gbase = wid * EGO_W
    pltpu.sync_copy(needed.at[pl.ds(gbase, EGO_W)], nbuf.at[pl.ds(0, EGO_W)])
    for r in range(EGO_W // G):
        o = gbase + r * G
        is_item = o >= B
        off = jnp.where(is_item, N_USER, 0)
        for k in range(G // 16):
            scolg[pl.ds(k * 16, 16)] = nbuf[pl.ds(r * G + k * 16, 16)] - off

        @pl.when(is_item)
        def _():
            pltpu.sync_copy(iemb.at[scolg.at[pl.ds(0, G)]],
                            rowsb.at[pl.ds(0, G)])

        @pl.when(jnp.logical_not(is_item))
        def _():
            pltpu.sync_copy(uemb.at[scolg.at[pl.ds(0, G)]],
                            rowsb.at[pl.ds(0, G)])

        pltpu.sync_copy(rowsb.at[pl.ds(0, G)], ego_out.at[pl.ds(o, G)])

    plsc.subcore_barrier()

    ebase = sid * EW

    def fire_edges(ci):
        b = (ci & 1) * C
        base = ebase + ci * C
        pltpu.async_copy(adj.at[0, pl.ds(base, C)], ebr.at[pl.ds(b, C)], esem)
        pltpu.async_copy(adj.at[1, pl.ds(base, C)], ebc.at[pl.ds(b, C)], esem)
        pltpu.async_copy(vals.at[pl.ds(base, C)], ebv.at[pl.ds(b, C)], esem)

    def wait_edges(ci):
        b = (ci & 1) * C
        base = ebase + ci * C
        pltpu.make_async_copy(adj.at[0, pl.ds(base, C)],
                              ebr.at[pl.ds(b, C)], esem).wait()
        pltpu.make_async_copy(adj.at[1, pl.ds(base, C)],
                              ebc.at[pl.ds(b, C)], esem).wait()
        pltpu.make_async_copy(vals.at[pl.ds(base, C)],
                              ebv.at[pl.ds(b, C)], esem).wait()

    fire_edges(jnp.int32(0))

    def chunk_body(ci, cnts):
        wait_edges(ci)

        @pl.when(ci + 1 < NCHUNK)
        def _():
            fire_edges(ci + 1)

        b = (ci & 1) * C

        def fbody(i, cnts):
            cu, cnt_i = cnts
            o = b + i * 16
            r16 = ebr[pl.ds(o, 16)] - lo
            inr = (r16 >= 0) & (r16 < HALF)
            s16 = plsc.load_gather(table, [jnp.clip(r16, 0, HALF - 1)])
            m = inr & (s16 >= 0)
            c16 = ebc[pl.ds(o, 16)]
            v16 = ebv[pl.ds(o, 16)]
            mu = m & (c16 < N_USER)
            mi = m & (c16 >= N_USER)
            pu = (c16 << SSH) | (s16 & (2**SSH - 1))
            pi = pu - (N_USER << SSH)
            plsc.store_compressed(spacku.at[pl.ds(cu, 16)], pu, mask=mu)
            plsc.store_compressed(svalu.at[pl.ds(cu, 16)], v16, mask=mu)
            plsc.store_compressed(spacki.at[pl.ds(cnt_i, 16)], pi, mask=mi)
            plsc.store_compressed(svali.at[pl.ds(cnt_i, 16)], v16, mask=mi)
            pcu = plsc.all_reduce_population_count(mu)
            pci = plsc.all_reduce_population_count(mi)
            return (jnp.minimum(cu + pcu[0], SCAP0),
                    jnp.minimum(cnt_i + pci[0], SCAP0))

        return lax.fori_loop(0, C // 16, fbody, cnts)

    cu, ci_cnt = lax.fori_loop(0, NCHUNK, chunk_body,
                               (jnp.int32(0), jnp.int32(0)))

    def gather_phase(spk, svl, cnt, emb_t):
        zi = jnp.zeros((16,), jnp.int32)
        zf = jnp.zeros((16,), jnp.float32)
        for k in range(G // 16):
            spk[pl.ds(cnt + k * 16, 16)] = zi
            svl[pl.ds(cnt + k * 16, 16)] = zf
        nb = (cnt + (G - 1)) // G

        def stage(g):
            gb = (g & 1) * G
            for k in range(8):
                p16 = spk[pl.ds(g * G + k * 16, 16)]
                scolg[pl.ds(gb + k * 16, 16)] = p16 >> SSH
                sslot2d[g & 1, pl.ds(k * 16, 16)] = p16 & (2**SSH - 1)

        def fire_gather(g):
            gb = (g & 1) * G
            pltpu.async_copy(emb_t.at[scolg.at[pl.ds(gb, G)]],
                             rowsb.at[pl.ds(gb, G)], gsem)

        def wait_gather(g):
            gb = (g & 1) * G
            pltpu.make_async_copy(emb_t.at[scolg.at[pl.ds(gb, G)]],
                                  rowsb.at[pl.ds(gb, G)], gsem).wait()

        def fire_scatter(g):
            gb = (g & 1) * G
            pltpu.async_copy(rowsb.at[pl.ds(gb, G)],
                             acc.at[sslot2d.at[g & 1]], ssem, add=True)

        def wait_scatter(g):
            gb = (g & 1) * G
            pltpu.make_async_copy(rowsb.at[pl.ds(gb, G)],
                                  acc.at[sslot2d.at[g & 1]], ssem).wait()

        @pl.when(nb > 0)
        def _():
            stage(jnp.int32(0))
            fire_gather(jnp.int32(0))

        def gloop(g, _):
            gb = (g & 1) * G
            wait_gather(g)

            @pl.when(g + 1 < nb)
            def _():
                @pl.when(g >= 1)
                def _():
                    wait_scatter(g - 1)

                stage(g + 1)
                fire_gather(g + 1)

            def scale16(bi, _):
                v16 = svl[pl.ds(g * G + bi * 16, 16)]
                for j in range(16):
                    i = gb + bi * 16 + j
                    v = v16[j]
                    rowsb[i, pl.ds(0, 16)] = rowsb[i, pl.ds(0, 16)] * v
                    rowsb[i, pl.ds(16, 16)] = rowsb[i, pl.ds(16, 16)] * v
                return 0

            lax.fori_loop(0, G // 16, scale16, 0)
            fire_scatter(g)
            return 0

        lax.fori_loop(0, nb, gloop, 0)

        @pl.when(nb >= 2)
        def _():
            wait_scatter(nb - 2)

        @pl.when(nb >= 1)
        def _():
            wait_scatter(nb - 1)

    gather_phase(spacku, svalu, cu, uemb)
    gather_phase(spacki, svali, ci_cnt, iemb)

    plsc.subcore_barrier()

    pbase = sid * OUT_T
    pltpu.sync_copy(needed.at[pl.ds(pbase, OUT_T)], nbuf)

    def slot_vreg(k, _):
        i16 = nbuf[pl.ds(k * 16, 16)] - lo
        inr = (i16 >= 0) & (i16 < HALF)
        s16 = plsc.load_gather(table, [jnp.clip(i16, 0, HALF - 1)])
        slotsb[pl.ds(k * 16, 16)] = jnp.where(inr & (s16 >= 0), s16,
                                              jnp.full((16,), NB3, jnp.int32))
        return 0

    lax.fori_loop(0, OUT_T // 16, slot_vreg, 0)
    obase = cid * NB3 + pbase
    for r in range(OUT_T // G):
        pltpu.sync_copy(acc.at[slotsb.at[pl.ds(r * G, G)]],
                        rowsb.at[pl.ds(0, G)])
        pltpu.sync_copy(rowsb.at[pl.ds(0, G)],
                        sidep_out.at[pl.ds(obase + r * G, G)])


_sc_call = functools.partial(
    pl.kernel,
    out_type=(
        jax.ShapeDtypeStruct((NB3, D), jnp.float32),
        jax.ShapeDtypeStruct((2 * NB3, D), jnp.float32),
    ),
    mesh=plsc.VectorSubcoreMesh(core_axis_name="c", subcore_axis_name="s"),
    compiler_params=pltpu.CompilerParams(needs_layout_passes=False,
                                         use_tc_tiling_on_sc=False),
    scratch_types=[
        pltpu.VMEM((TBL,), jnp.int32),
        pltpu.VMEM((2 * C,), jnp.int32),
        pltpu.VMEM((2 * C,), jnp.int32),
        pltpu.VMEM((2 * C,), jnp.float32),
        pltpu.VMEM((SCAPB,), jnp.int32),
        pltpu.VMEM((SCAPB,), jnp.float32),
        pltpu.VMEM((SCAPB,), jnp.int32),
        pltpu.VMEM((SCAPB,), jnp.float32),
        pltpu.VMEM((2 * G,), jnp.int32),
        pltpu.VMEM((2, G), jnp.int32),
        pltpu.VMEM((2 * G, D), jnp.float32),
        pltpu.VMEM((OUT_T,), jnp.int32),
        pltpu.VMEM((OUT_T,), jnp.int32),
        pltpu.VMEM_SHARED((ACC_R, D), jnp.float32),
        pltpu.SemaphoreType.DMA,
        pltpu.SemaphoreType.DMA,
        pltpu.SemaphoreType.DMA,
    ],
)(_sc_body)


def _tc_body(ego_ref, sidep_ref, wg0, bg0, wb0, bb0, wg1, bg1, wb1, bb1,
             u_ref, p_ref, n_ref):
    side = sidep_ref[0:NB3, :] + sidep_ref[NB3:2 * NB3, :]
    e0 = ego_ref[...]

    def layer(ego, wg, bg, wb, bb):
        sum_emb = jnp.dot(side, wg[...],
                          preferred_element_type=jnp.float32) + bg[...]
        bi_emb = jnp.dot(ego * side, wb[...],
                         preferred_element_type=jnp.float32) + bb[...]
        x = sum_emb + bi_emb
        x = jnp.where(x >= 0, x, 0.2 * x)
        nrm = jnp.sqrt(jnp.sum(x * x, axis=1, keepdims=True))
        return x, x / jnp.maximum(nrm, 1e-12)

    e1, n1 = layer(e0, wg0, bg0, wb0, bb0)
    e2, n2 = layer(e1, wg1, bg1, wb1, bb1)
    allemb = jnp.concatenate([e0, n1, n2], axis=1)
    u_ref[...] = allemb[0:B]
    p_ref[...] = allemb[B:2 * B]
    n_ref[...] = allemb[2 * B:3 * B]


def _tc_epilogue(ego, sidep, wg0, bg0, wb0, bb0, wg1, bg1, wb1, bb1):
    return pl.pallas_call(
        _tc_body,
        out_shape=[jax.ShapeDtypeStruct((B, 3 * D), jnp.float32)] * 3,
    )(ego, sidep, wg0, bg0, wb0, bb0, wg1, bg1, wb1, bb1)


def kernel(users, pos_items, neg_items, adj_indices, adj_values, user_emb,
           item_emb, W_gc_0, b_gc_0, W_bi_0, b_bi_0, W_gc_1, b_gc_1, W_bi_1,
           b_bi_1):
    needed = jnp.concatenate(
        [users, pos_items + N_USER, neg_items + N_USER]).astype(jnp.int32)
    neg1 = jnp.full((TBL,), -1, jnp.int32)
    zacc = jnp.zeros((ACC_R, D), jnp.float32)
    ego, sidep = _sc_call(needed, adj_indices, adj_values, user_emb,
                          item_emb, neg1, zacc)
    return _tc_epilogue(ego, sidep, W_gc_0, b_gc_0, W_bi_0, b_bi_0,
                        W_gc_1, b_gc_1, W_bi_1, b_bi_1)
